# Initial kernel scaffold; baseline (speedup 1.0000x reference)
import functools

import jax
import jax.numpy as jnp
from jax import lax
from jax.experimental import pallas as pl
from jax.experimental.pallas import tpu as pltpu
from jax.experimental.pallas import tpu_sc as plsc

N = 10000
E = 320000
D = 128

NC = 2
NS = 16
L = 16
NW = NC * NS
EPW = E // NW
C = 80
NCHUNK = EPW // C
ROWS_PT = N // NS

_BN = 1000
_BE = 2000


def _node_pre_body(x_ref, wsrc_ref, bsrc_ref, wdst_ref, bdst_ref,
                   wself_ref, bias_ref, esrc_ref, edst_ref, selfp_ref):
    x = x_ref[...]
    esrc_ref[...] = jnp.dot(x, wsrc_ref[...],
                            preferred_element_type=jnp.float32) + bsrc_ref[...]
    edst_ref[...] = jnp.dot(x, wdst_ref[...],
                            preferred_element_type=jnp.float32) + bdst_ref[...]
    selfp_ref[...] = (jnp.dot(x, wself_ref[...],
                              preferred_element_type=jnp.float32)
                      + bias_ref[...] + x)


def _node_pre(node_feats, wsrc_t, bsrc, wdst_t, bdst, wself_t, bias):
    f32 = jnp.float32
    row = pl.BlockSpec((_BN, D), lambda i: (i, 0))
    mat = pl.BlockSpec((D, D), lambda i: (0, 0))
    vec = pl.BlockSpec((1, D), lambda i: (0, 0))
    return pl.pallas_call(
        _node_pre_body,
        grid=(N // _BN,),
        in_specs=[row, mat, vec, mat, vec, mat, vec],
        out_specs=[row, row, row],
        out_shape=[jax.ShapeDtypeStruct((N, D), f32)] * 3,
    )(node_feats, wsrc_t, bsrc, wdst_t, bdst, wself_t, bias)


def _edge_pre_body(x_ref, w_ref, b_ref, out_ref):
    out_ref[...] = (jnp.dot(x_ref[...], w_ref[...],
                            preferred_element_type=jnp.float32) + b_ref[...])


def _edge_pre(edge_feats, wedge_t, bedge):
    row = pl.BlockSpec((_BE, D), lambda i: (i, 0))
    return pl.pallas_call(
        _edge_pre_body,
        grid=(E // _BE,),
        in_specs=[row,
                  pl.BlockSpec((D, D), lambda i: (0, 0)),
                  pl.BlockSpec((1, D), lambda i: (0, 0))],
        out_specs=row,
        out_shape=jax.ShapeDtypeStruct((E, D), jnp.float32),
    )(edge_feats, wedge_t, bedge)


def _sc_body(src_hbm, dst_hbm, esrc_hbm, edst_hbm, nf_hbm, ee_hbm,
             zrow_hbm, zdeg_hbm, ones_hbm,
             m_hbm, sum_hbm, deg_hbm,
             sidx, didx, gsrc, gdst, gnf, gee, ones_v, acc, dacc,
             s1, s2, s3, s4):
    cid = lax.axis_index("c")
    sid = lax.axis_index("s")
    wid = sid * NC + cid
    base0 = pl.multiple_of(wid * EPW, 8)
    row0 = pl.multiple_of(sid * ROWS_PT, 8)

    pltpu.sync_copy(zrow_hbm.at[pl.ds(row0, ROWS_PT)],
                    acc.at[pl.ds(row0, ROWS_PT)])
    pltpu.sync_copy(zdeg_hbm.at[pl.ds(row0, ROWS_PT)],
                    dacc.at[pl.ds(row0, ROWS_PT)])
    pltpu.sync_copy(ones_hbm, ones_v)
    plsc.subcore_barrier()

    @pl.loop(0, NCHUNK)
    def _chunk(k):
        base = pl.multiple_of(base0 + k * C, 8)
        pltpu.sync_copy(src_hbm.at[pl.ds(base, C)], sidx)
        pltpu.sync_copy(dst_hbm.at[pl.ds(base, C)], didx)
        c1 = pltpu.async_copy(esrc_hbm.at[sidx], gsrc, s1)
        c2 = pltpu.async_copy(edst_hbm.at[didx], gdst, s2)
        c3 = pltpu.async_copy(nf_hbm.at[sidx], gnf, s3)
        c4 = pltpu.async_copy(ee_hbm.at[pl.ds(base, C)], gee, s4)
        c1.wait()
        c2.wait()
        c3.wait()
        c4.wait()

        @pl.loop(0, C)
        def _row(r):
            for c in range(0, D, L):
                sl = pl.ds(c, L)
                mval = gsrc[r, sl] + gdst[r, sl] + gee[r, sl]
                gee[r, sl] = mval
                sig = 1.0 / (1.0 + jnp.exp(-mval))
                gnf[r, sl] = gnf[r, sl] * sig

        pltpu.sync_copy(gee, m_hbm.at[pl.ds(base, C)])
        pltpu.sync_copy(gnf, acc.at[didx], add=True)
        pltpu.sync_copy(ones_v, dacc.at[didx], add=True)

    plsc.subcore_barrier()
    pltpu.sync_copy(acc.at[pl.ds(row0, ROWS_PT)],
                    sum_hbm.at[cid, pl.ds(row0, ROWS_PT)])
    pltpu.sync_copy(dacc.at[pl.ds(row0, ROWS_PT)],
                    deg_hbm.at[cid, pl.ds(row0, ROWS_PT)])


def _sc_gather_scatter(src, dst, e_src, e_dst, node_feats, e_edge,
                       zrow, zdeg, ones):
    f32 = jnp.float32
    fn = pl.kernel(
        _sc_body,
        out_type=(jax.ShapeDtypeStruct((E, D), f32),
                  jax.ShapeDtypeStruct((NC, N, D), f32),
                  jax.ShapeDtypeStruct((NC, N, L), f32)),
        mesh=plsc.VectorSubcoreMesh(core_axis_name="c", subcore_axis_name="s"),
        scratch_types=[
            pltpu.VMEM((C,), jnp.int32),
            pltpu.VMEM((C,), jnp.int32),
            pltpu.VMEM((C, D), f32),
            pltpu.VMEM((C, D), f32),
            pltpu.VMEM((C, D), f32),
            pltpu.VMEM((C, D), f32),
            pltpu.VMEM((C, L), f32),
            pltpu.VMEM_SHARED((N, D), f32),
            pltpu.VMEM_SHARED((N, L), f32),
            pltpu.SemaphoreType.DMA,
            pltpu.SemaphoreType.DMA,
            pltpu.SemaphoreType.DMA,
            pltpu.SemaphoreType.DMA,
        ],
    )
    return fn(src, dst, e_src, e_dst, node_feats, e_edge, zrow, zdeg, ones)


def _combine_body(sum_ref, deg_ref, selfp_ref, wneigh_ref, out_ref):
    s = sum_ref[0] + sum_ref[1]
    dg = deg_ref[0, :, 0:1] + deg_ref[1, :, 0:1]
    h = s / jnp.maximum(dg, 1.0)
    out_ref[...] = selfp_ref[...] + jnp.dot(
        h, wneigh_ref[...], preferred_element_type=jnp.float32)


def _combine(sums, degs, selfp, wneigh_t):
    row = pl.BlockSpec((_BN, D), lambda i: (i, 0))
    return pl.pallas_call(
        _combine_body,
        grid=(N // _BN,),
        in_specs=[pl.BlockSpec((NC, _BN, D), lambda i: (0, i, 0)),
                  pl.BlockSpec((NC, _BN, L), lambda i: (0, i, 0)),
                  row,
                  pl.BlockSpec((D, D), lambda i: (0, 0))],
        out_specs=row,
        out_shape=jax.ShapeDtypeStruct((N, D), jnp.float32),
    )(sums, degs, selfp, wneigh_t)


def kernel(node_feats, edge_index, edge_feats,
           W_src_gate, b_src_gate, W_dst_gate, b_dst_gate,
           W_edge_gate, b_edge_gate, W_self, W_neigh, bias):
    src = edge_index[0].astype(jnp.int32)
    dst = edge_index[1].astype(jnp.int32)

    e_src, e_dst, selfp = _node_pre(
        node_feats, W_src_gate.T, b_src_gate.reshape(1, D),
        W_dst_gate.T, b_dst_gate.reshape(1, D),
        W_self.T, bias.reshape(1, D))
    e_edge = _edge_pre(edge_feats, W_edge_gate.T, b_edge_gate.reshape(1, D))

    zrow = jnp.zeros((N, D), jnp.float32)
    zdeg = jnp.zeros((N, L), jnp.float32)
    ones = jnp.ones((C, L), jnp.float32)
    m, sums, degs = _sc_gather_scatter(src, dst, e_src, e_dst, node_feats,
                                       e_edge, zrow, zdeg, ones)

    rst = _combine(sums, degs, selfp, W_neigh.T)
    return (rst, m)


# speedup vs baseline: 3.5945x; 3.5945x over previous
import functools

import jax
import jax.numpy as jnp
from jax import lax
from jax.experimental import pallas as pl
from jax.experimental.pallas import tpu as pltpu
from jax.experimental.pallas import tpu_sc as plsc

N = 10000
E = 320000
D = 128

NC = 2
NS = 16
L = 16
NW = NC * NS
EPW = E // NW
C = 80
NCHUNK = EPW // C
NP = 10240
ROWS_PT = NP // NS

_BN = 1000
_BE = 2000


def _node_pre_body(x_ref, wsrc_ref, bsrc_ref, wdst_ref, bdst_ref,
                   wself_ref, bias_ref, esrc_ref, edst_ref, selfp_ref):
    x = x_ref[...]
    esrc_ref[...] = jnp.dot(x, wsrc_ref[...],
                            preferred_element_type=jnp.float32) + bsrc_ref[...]
    edst_ref[...] = jnp.dot(x, wdst_ref[...],
                            preferred_element_type=jnp.float32) + bdst_ref[...]
    selfp_ref[...] = (jnp.dot(x, wself_ref[...],
                              preferred_element_type=jnp.float32)
                      + bias_ref[...] + x)


def _node_pre(node_feats, wsrc_t, bsrc, wdst_t, bdst, wself_t, bias):
    f32 = jnp.float32
    row = pl.BlockSpec((_BN, D), lambda i: (i, 0))
    mat = pl.BlockSpec((D, D), lambda i: (0, 0))
    vec = pl.BlockSpec((1, D), lambda i: (0, 0))
    return pl.pallas_call(
        _node_pre_body,
        grid=(N // _BN,),
        in_specs=[row, mat, vec, mat, vec, mat, vec],
        out_specs=[row, row, row],
        out_shape=[jax.ShapeDtypeStruct((N, D), f32)] * 3,
    )(node_feats, wsrc_t, bsrc, wdst_t, bdst, wself_t, bias)


def _edge_pre_body(x_ref, w_ref, b_ref, out_ref):
    out_ref[...] = (jnp.dot(x_ref[...], w_ref[...],
                            preferred_element_type=jnp.float32) + b_ref[...])


def _edge_pre(edge_feats, wedge_t, bedge):
    row = pl.BlockSpec((_BE, D), lambda i: (i, 0))
    return pl.pallas_call(
        _edge_pre_body,
        grid=(E // _BE,),
        in_specs=[row,
                  pl.BlockSpec((D, D), lambda i: (0, 0)),
                  pl.BlockSpec((1, D), lambda i: (0, 0))],
        out_specs=row,
        out_shape=jax.ShapeDtypeStruct((E, D), jnp.float32),
    )(edge_feats, wedge_t, bedge)


def _sc_body(src_hbm, dst_hbm, esrc_hbm, edst_hbm, nf_hbm, ee_hbm,
             zrow_hbm,
             m_hbm, sum_hbm,
             sidx, didx, gsrc, gdst, gnf, gee, acc,
             s1, s2, s3, s4):
    cid = lax.axis_index("c")
    sid = lax.axis_index("s")
    wid = sid * NC + cid
    base0 = pl.multiple_of(wid * EPW, 8)
    row0 = pl.multiple_of(sid * ROWS_PT, 8)

    pltpu.sync_copy(zrow_hbm.at[pl.ds(row0, ROWS_PT)],
                    acc.at[pl.ds(row0, ROWS_PT)])
    plsc.subcore_barrier()

    @pl.loop(0, NCHUNK)
    def _chunk(k):
        base = pl.multiple_of(base0 + k * C, 8)
        pltpu.sync_copy(src_hbm.at[pl.ds(base, C)], sidx)
        pltpu.sync_copy(dst_hbm.at[pl.ds(base, C)], didx)
        c1 = pltpu.async_copy(esrc_hbm.at[sidx], gsrc, s1)
        c2 = pltpu.async_copy(edst_hbm.at[didx], gdst, s2)
        c3 = pltpu.async_copy(nf_hbm.at[sidx], gnf, s3)
        c4 = pltpu.async_copy(ee_hbm.at[pl.ds(base, C)], gee, s4)
        c1.wait()
        c2.wait()
        c3.wait()
        c4.wait()

        @pl.loop(0, C)
        def _row(r):
            for c in range(0, D, L):
                sl = pl.ds(c, L)
                mval = gsrc[r, sl] + gdst[r, sl] + gee[r, sl]
                gee[r, sl] = mval
                sig = 1.0 / (1.0 + jnp.exp(-mval))
                gnf[r, sl] = gnf[r, sl] * sig

        pltpu.sync_copy(gee, m_hbm.at[pl.ds(base, C)])
        pltpu.sync_copy(gnf, acc.at[didx], add=True)

    plsc.subcore_barrier()
    pltpu.sync_copy(acc.at[pl.ds(row0, ROWS_PT)],
                    sum_hbm.at[cid, pl.ds(row0, ROWS_PT)])


def _sc_gather_scatter(src, dst, e_src, e_dst, node_feats, e_edge, zrow):
    f32 = jnp.float32
    fn = pl.kernel(
        _sc_body,
        out_type=(jax.ShapeDtypeStruct((E, D), f32),
                  jax.ShapeDtypeStruct((NC, NP, D), f32)),
        mesh=plsc.VectorSubcoreMesh(core_axis_name="c", subcore_axis_name="s"),
        scratch_types=[
            pltpu.VMEM((C,), jnp.int32),
            pltpu.VMEM((C,), jnp.int32),
            pltpu.VMEM((C, D), f32),
            pltpu.VMEM((C, D), f32),
            pltpu.VMEM((C, D), f32),
            pltpu.VMEM((C, D), f32),
            pltpu.VMEM_SHARED((NP, D), f32),
            pltpu.SemaphoreType.DMA,
            pltpu.SemaphoreType.DMA,
            pltpu.SemaphoreType.DMA,
            pltpu.SemaphoreType.DMA,
        ],
    )
    return fn(src, dst, e_src, e_dst, node_feats, e_edge, zrow)


def _sc_deg_body(dst_hbm, zdeg_hbm, ones_hbm, deg_hbm, didx, ones_v, dacc):
    cid = lax.axis_index("c")
    sid = lax.axis_index("s")
    wid = sid * NC + cid
    base0 = pl.multiple_of(wid * EPW, 8)
    row0 = pl.multiple_of(sid * ROWS_PT, 8)

    pltpu.sync_copy(zdeg_hbm.at[pl.ds(row0, ROWS_PT)],
                    dacc.at[pl.ds(row0, ROWS_PT)])
    pltpu.sync_copy(ones_hbm, ones_v)
    plsc.subcore_barrier()

    @pl.loop(0, NCHUNK)
    def _chunk(k):
        base = pl.multiple_of(base0 + k * C, 8)
        pltpu.sync_copy(dst_hbm.at[pl.ds(base, C)], didx)
        pltpu.sync_copy(ones_v, dacc.at[didx], add=True)

    plsc.subcore_barrier()
    pltpu.sync_copy(dacc.at[pl.ds(row0, ROWS_PT)],
                    deg_hbm.at[cid, pl.ds(row0, ROWS_PT)])


def _sc_degree(dst, zdeg, ones):
    f32 = jnp.float32
    fn = pl.kernel(
        _sc_deg_body,
        out_type=jax.ShapeDtypeStruct((NC, NP, D), f32),
        mesh=plsc.VectorSubcoreMesh(core_axis_name="c", subcore_axis_name="s"),
        scratch_types=[
            pltpu.VMEM((C,), jnp.int32),
            pltpu.VMEM((C, D), f32),
            pltpu.VMEM_SHARED((NP, D), f32),
        ],
    )
    return fn(dst, zdeg, ones)


def _combine_body(sum_ref, deg_ref, selfp_ref, wneigh_ref, out_ref):
    s = sum_ref[0] + sum_ref[1]
    dg = deg_ref[0, :, 0:1] + deg_ref[1, :, 0:1]
    h = s / jnp.maximum(dg, 1.0)
    out_ref[...] = selfp_ref[...] + jnp.dot(
        h, wneigh_ref[...], preferred_element_type=jnp.float32)


def _combine(sums, degs, selfp, wneigh_t):
    row = pl.BlockSpec((_BN, D), lambda i: (i, 0))
    return pl.pallas_call(
        _combine_body,
        grid=(N // _BN,),
        in_specs=[pl.BlockSpec((NC, _BN, D), lambda i: (0, i, 0)),
                  pl.BlockSpec((NC, _BN, D), lambda i: (0, i, 0)),
                  row,
                  pl.BlockSpec((D, D), lambda i: (0, 0))],
        out_specs=row,
        out_shape=jax.ShapeDtypeStruct((N, D), jnp.float32),
    )(sums, degs, selfp, wneigh_t)


def kernel(node_feats, edge_index, edge_feats,
           W_src_gate, b_src_gate, W_dst_gate, b_dst_gate,
           W_edge_gate, b_edge_gate, W_self, W_neigh, bias):
    src = edge_index[0].astype(jnp.int32)
    dst = edge_index[1].astype(jnp.int32)

    e_src, e_dst, selfp = _node_pre(
        node_feats, W_src_gate.T, b_src_gate.reshape(1, D),
        W_dst_gate.T, b_dst_gate.reshape(1, D),
        W_self.T, bias.reshape(1, D))
    e_edge = _edge_pre(edge_feats, W_edge_gate.T, b_edge_gate.reshape(1, D))

    zrow = jnp.zeros((NP, D), jnp.float32)
    ones = jnp.ones((C, D), jnp.float32)
    degs = _sc_degree(dst, zrow, ones)
    m, sums = _sc_gather_scatter(src, dst, e_src, e_dst, node_feats,
                                 e_edge, zrow)

    rst = _combine(sums, degs, selfp, W_neigh.T)
    return (rst, m)


# speedup vs baseline: 5.3216x; 1.4805x over previous
import functools

import jax
import jax.numpy as jnp
from jax import lax
from jax.experimental import pallas as pl
from jax.experimental.pallas import tpu as pltpu
from jax.experimental.pallas import tpu_sc as plsc

N = 10000
E = 320000
D = 128

NC = 2
NS = 16
L = 16
NW = NC * NS
EPW = E // NW
C = 80
CDEG = EPW // C
CH = 40
NCHUNK = EPW // CH
NP = 10240
ROWS_PT = NP // NS

_BN = 1000
_BE = 2000


def _node_pre_body(x_ref, wsrc_ref, bsrc_ref, wdst_ref, bdst_ref,
                   wself_ref, bias_ref, esrc_ref, edst_ref, selfp_ref):
    x = x_ref[...]
    esrc_ref[...] = jnp.dot(x, wsrc_ref[...],
                            preferred_element_type=jnp.float32) + bsrc_ref[...]
    edst_ref[...] = jnp.dot(x, wdst_ref[...],
                            preferred_element_type=jnp.float32) + bdst_ref[...]
    selfp_ref[...] = (jnp.dot(x, wself_ref[...],
                              preferred_element_type=jnp.float32)
                      + bias_ref[...] + x)


def _node_pre(node_feats, wsrc_t, bsrc, wdst_t, bdst, wself_t, bias):
    f32 = jnp.float32
    row = pl.BlockSpec((_BN, D), lambda i: (i, 0))
    mat = pl.BlockSpec((D, D), lambda i: (0, 0))
    vec = pl.BlockSpec((1, D), lambda i: (0, 0))
    return pl.pallas_call(
        _node_pre_body,
        grid=(N // _BN,),
        in_specs=[row, mat, vec, mat, vec, mat, vec],
        out_specs=[row, row, row],
        out_shape=[jax.ShapeDtypeStruct((N, D), f32)] * 3,
    )(node_feats, wsrc_t, bsrc, wdst_t, bdst, wself_t, bias)


def _edge_pre_body(x_ref, w_ref, b_ref, out_ref):
    out_ref[...] = (jnp.dot(x_ref[...], w_ref[...],
                            preferred_element_type=jnp.float32) + b_ref[...])


def _edge_pre(edge_feats, wedge_t, bedge):
    row = pl.BlockSpec((_BE, D), lambda i: (i, 0))
    return pl.pallas_call(
        _edge_pre_body,
        grid=(E // _BE,),
        in_specs=[row,
                  pl.BlockSpec((D, D), lambda i: (0, 0)),
                  pl.BlockSpec((1, D), lambda i: (0, 0))],
        out_specs=row,
        out_shape=jax.ShapeDtypeStruct((E, D), jnp.float32),
    )(edge_feats, wedge_t, bedge)


def _sc_body(src_hbm, dst_hbm, esrc_hbm, edst_hbm, nf_hbm, ee_hbm,
             zrow_hbm,
             m_hbm, sum_hbm,
             sidx0, didx0, sidx1, didx1, dscat0, dscat1,
             g0src, g0dst, g0nf, g0ee, g1src, g1dst, g1nf, g1ee,
             acc, gs0, gs1, ws0, ws1, is0, is1):
    cid = lax.axis_index("c")
    sid = lax.axis_index("s")
    wid = sid * NC + cid
    base0 = pl.multiple_of(wid * EPW, 8)
    row0 = pl.multiple_of(sid * ROWS_PT, 8)

    pltpu.sync_copy(zrow_hbm.at[pl.ds(row0, ROWS_PT)],
                    acc.at[pl.ds(row0, ROWS_PT)])
    plsc.subcore_barrier()

    bufs = ((sidx0, didx0, g0src, g0dst, g0nf, g0ee, gs0, ws0, is0, dscat0),
            (sidx1, didx1, g1src, g1dst, g1nf, g1ee, gs1, ws1, is1, dscat1))

    def snapshot_scatter_idx(b):
        di, dsc = bufs[b][1], bufs[b][9]
        for off in (0, 16, 24):
            dsc[pl.ds(off, L)] = di[pl.ds(off, L)]

    def fire_idx(b, k):
        si, di, isem = bufs[b][0], bufs[b][1], bufs[b][8]
        base = pl.multiple_of(base0 + k * CH, 8)
        pltpu.async_copy(src_hbm.at[pl.ds(base, CH)], si, isem)
        pltpu.async_copy(dst_hbm.at[pl.ds(base, CH)], di, isem)

    def drain_idx(b, k):
        si, di, isem = bufs[b][0], bufs[b][1], bufs[b][8]
        base = pl.multiple_of(base0 + k * CH, 8)
        pltpu.make_async_copy(src_hbm.at[pl.ds(base, CH)], si, isem).wait()
        pltpu.make_async_copy(dst_hbm.at[pl.ds(base, CH)], di, isem).wait()

    def fire_gathers(b, k):
        si, di, gsrc, gdst, gnf, gee, gsem = bufs[b][:7]
        base = pl.multiple_of(base0 + k * CH, 8)
        pltpu.async_copy(esrc_hbm.at[si], gsrc, gsem)
        pltpu.async_copy(edst_hbm.at[di], gdst, gsem)
        pltpu.async_copy(nf_hbm.at[si], gnf, gsem)
        pltpu.async_copy(ee_hbm.at[pl.ds(base, CH)], gee, gsem)

    def drain_gathers(b, k):
        si, di, gsrc, gdst, gnf, gee, gsem = bufs[b][:7]
        base = pl.multiple_of(base0 + k * CH, 8)
        pltpu.make_async_copy(esrc_hbm.at[si], gsrc, gsem).wait()
        pltpu.make_async_copy(edst_hbm.at[di], gdst, gsem).wait()
        pltpu.make_async_copy(nf_hbm.at[si], gnf, gsem).wait()
        pltpu.make_async_copy(ee_hbm.at[pl.ds(base, CH)], gee, gsem).wait()

    def compute(b):
        gsrc, gdst, gnf, gee = bufs[b][2:6]

        @pl.loop(0, CH)
        def _row(r):
            for c in range(0, D, L):
                sl = pl.ds(c, L)
                mval = gsrc[r, sl] + gdst[r, sl] + gee[r, sl]
                gee[r, sl] = mval
                sig = 1.0 / (1.0 + jnp.exp(-mval))
                gnf[r, sl] = gnf[r, sl] * sig

    def fire_m_write(b, k):
        gee, wsem = bufs[b][5], bufs[b][7]
        base = pl.multiple_of(base0 + k * CH, 8)
        pltpu.async_copy(gee, m_hbm.at[pl.ds(base, CH)], wsem)

    def drain_m_write(b, k):
        gee, wsem = bufs[b][5], bufs[b][7]
        base = pl.multiple_of(base0 + k * CH, 8)
        pltpu.make_async_copy(gee, m_hbm.at[pl.ds(base, CH)], wsem).wait()

    def scatter_msg(b):
        gnf, dsc = bufs[b][4], bufs[b][9]
        pltpu.sync_copy(gnf, acc.at[dsc], add=True)

    def step(b, k, last):
        drain_gathers(b, k)
        snapshot_scatter_idx(b)
        if not last:
            fire_idx(b, k + 2)
        compute(b)
        fire_m_write(b, k)
        scatter_msg(b)
        drain_m_write(b, k)
        if not last:
            drain_idx(b, k + 2)
            fire_gathers(b, k + 2)

    fire_idx(0, 0)
    fire_idx(1, 1)
    drain_idx(0, 0)
    fire_gathers(0, 0)
    drain_idx(1, 1)
    fire_gathers(1, 1)

    @pl.loop(0, NCHUNK // 2 - 1)
    def _pair(t):
        step(0, 2 * t, False)
        step(1, 2 * t + 1, False)

    step(0, NCHUNK - 2, True)
    step(1, NCHUNK - 1, True)

    plsc.subcore_barrier()
    pltpu.sync_copy(acc.at[pl.ds(row0, ROWS_PT)],
                    sum_hbm.at[cid, pl.ds(row0, ROWS_PT)])


def _sc_gather_scatter(src, dst, e_src, e_dst, node_feats, e_edge, zrow):
    f32 = jnp.float32
    i32 = jnp.int32
    fn = pl.kernel(
        _sc_body,
        out_type=(jax.ShapeDtypeStruct((E, D), f32),
                  jax.ShapeDtypeStruct((NC, NP, D), f32)),
        mesh=plsc.VectorSubcoreMesh(core_axis_name="c", subcore_axis_name="s"),
        scratch_types=[
            pltpu.VMEM((CH,), i32),
            pltpu.VMEM((CH,), i32),
            pltpu.VMEM((CH,), i32),
            pltpu.VMEM((CH,), i32),
            pltpu.VMEM((CH,), i32),
            pltpu.VMEM((CH,), i32),
            pltpu.VMEM((CH, D), f32),
            pltpu.VMEM((CH, D), f32),
            pltpu.VMEM((CH, D), f32),
            pltpu.VMEM((CH, D), f32),
            pltpu.VMEM((CH, D), f32),
            pltpu.VMEM((CH, D), f32),
            pltpu.VMEM((CH, D), f32),
            pltpu.VMEM((CH, D), f32),
            pltpu.VMEM_SHARED((NP, D), f32),
            pltpu.SemaphoreType.DMA,
            pltpu.SemaphoreType.DMA,
            pltpu.SemaphoreType.DMA,
            pltpu.SemaphoreType.DMA,
            pltpu.SemaphoreType.DMA,
            pltpu.SemaphoreType.DMA,
        ],
    )
    return fn(src, dst, e_src, e_dst, node_feats, e_edge, zrow)


def _sc_deg_body(dst_hbm, zdeg_hbm, ones_hbm, deg_hbm, didx, ones_v, dacc):
    cid = lax.axis_index("c")
    sid = lax.axis_index("s")
    wid = sid * NC + cid
    base0 = pl.multiple_of(wid * EPW, 8)
    row0 = pl.multiple_of(sid * ROWS_PT, 8)

    pltpu.sync_copy(zdeg_hbm.at[pl.ds(row0, ROWS_PT)],
                    dacc.at[pl.ds(row0, ROWS_PT)])
    pltpu.sync_copy(ones_hbm, ones_v)
    plsc.subcore_barrier()

    @pl.loop(0, CDEG)
    def _chunk(k):
        base = pl.multiple_of(base0 + k * C, 8)
        pltpu.sync_copy(dst_hbm.at[pl.ds(base, C)], didx)
        pltpu.sync_copy(ones_v, dacc.at[didx], add=True)

    plsc.subcore_barrier()
    pltpu.sync_copy(dacc.at[pl.ds(row0, ROWS_PT)],
                    deg_hbm.at[cid, pl.ds(row0, ROWS_PT)])


def _sc_degree(dst, zdeg, ones):
    f32 = jnp.float32
    fn = pl.kernel(
        _sc_deg_body,
        out_type=jax.ShapeDtypeStruct((NC, NP, D), f32),
        mesh=plsc.VectorSubcoreMesh(core_axis_name="c", subcore_axis_name="s"),
        scratch_types=[
            pltpu.VMEM((C,), jnp.int32),
            pltpu.VMEM((C, D), f32),
            pltpu.VMEM_SHARED((NP, D), f32),
        ],
    )
    return fn(dst, zdeg, ones)


def _combine_body(sum_ref, deg_ref, selfp_ref, wneigh_ref, out_ref):
    s = sum_ref[0] + sum_ref[1]
    dg = deg_ref[0, :, 0:1] + deg_ref[1, :, 0:1]
    h = s / jnp.maximum(dg, 1.0)
    out_ref[...] = selfp_ref[...] + jnp.dot(
        h, wneigh_ref[...], preferred_element_type=jnp.float32)


def _combine(sums, degs, selfp, wneigh_t):
    row = pl.BlockSpec((_BN, D), lambda i: (i, 0))
    return pl.pallas_call(
        _combine_body,
        grid=(N // _BN,),
        in_specs=[pl.BlockSpec((NC, _BN, D), lambda i: (0, i, 0)),
                  pl.BlockSpec((NC, _BN, D), lambda i: (0, i, 0)),
                  row,
                  pl.BlockSpec((D, D), lambda i: (0, 0))],
        out_specs=row,
        out_shape=jax.ShapeDtypeStruct((N, D), jnp.float32),
    )(sums, degs, selfp, wneigh_t)


def kernel(node_feats, edge_index, edge_feats,
           W_src_gate, b_src_gate, W_dst_gate, b_dst_gate,
           W_edge_gate, b_edge_gate, W_self, W_neigh, bias):
    src = edge_index[0].astype(jnp.int32)
    dst = edge_index[1].astype(jnp.int32)

    e_src, e_dst, selfp = _node_pre(
        node_feats, W_src_gate.T, b_src_gate.reshape(1, D),
        W_dst_gate.T, b_dst_gate.reshape(1, D),
        W_self.T, bias.reshape(1, D))
    e_edge = _edge_pre(edge_feats, W_edge_gate.T, b_edge_gate.reshape(1, D))

    zrow = jnp.zeros((NP, D), jnp.float32)
    ones = jnp.ones((C, D), jnp.float32)
    degs = _sc_degree(dst, zrow, ones)
    m, sums = _sc_gather_scatter(src, dst, e_src, e_dst, node_feats,
                                 e_edge, zrow)

    rst = _combine(sums, degs, selfp, W_neigh.T)
    return (rst, m)
